# Initial kernel scaffold; baseline (speedup 1.0000x reference)
import jax
import jax.numpy as jnp
from jax.experimental import pallas as pl


def kernel(x, edge_index, W_in, b_in, W_c1, b_c1, g1, be1, W_c2, b_c2, g2, be2, W_out, b_out):
    raise NotImplementedError("write your pallas kernel here")


# speedup vs baseline: 10.4780x; 10.4780x over previous
import functools

import jax
import jax.numpy as jnp
from jax import lax
from jax.experimental import pallas as pl
from jax.experimental.pallas import tpu as pltpu
from jax.experimental.pallas import tpu_sc as plsc

_N = 10000
_D = 128
_E = 320000

_NC = 2
_NS = 16
_NW = _NC * _NS
_CH = 128
_NCH = (-(-_E // (_NW * _CH)) + 7) // 8 * 8
_EPAD = _NW * _CH * _NCH
_HC = _NCH // 2
_NP = 10240
_RPS = _NP // _NS
_RB = 2000


def _deg_body(dst_hbm, out_hbm, idx_d, ones_v, zb, deg, dsem):
    cid = lax.axis_index("c")
    sid = lax.axis_index("s")
    wid = sid * _NC + cid

    def _z16(i, c):
        zb[pl.ds(i * 16, 16)] = jnp.zeros((16,), jnp.float32)
        return c

    lax.fori_loop(0, _RPS // 16, _z16, 0)
    for j in range(_CH // 16):
        ones_v[pl.ds(j * 16, 16)] = jnp.ones((16,), jnp.float32)

    base = sid * _RPS
    pltpu.sync_copy(zb, deg.at[pl.ds(base, _RPS)])
    pltpu.sync_copy(dst_hbm.at[pl.ds(wid * _NCH, _NCH)], idx_d)
    plsc.subcore_barrier()

    hs = []
    for g in range(_NCH):
        hs.append(pltpu.async_copy(ones_v, deg.at[idx_d.at[g]], dsem, add=True))
    for h in hs:
        h.wait()

    plsc.subcore_barrier()
    pltpu.sync_copy(deg.at[pl.ds(base, _RPS)],
                    out_hbm.at[pl.ds(cid * _NP + base, _RPS)])


def _conv_body(ts_hbm, src_hbm, dst_hbm, out_hbm,
               idx_s, idx_d, rows0, rows1, acc, gs0, gs1, ss0, ss1):
    cid = lax.axis_index("c")
    sid = lax.axis_index("s")
    wid = sid * _NC + cid

    def _zrow(i, c):
        for j in range(_D // 16):
            rows0[i, pl.ds(j * 16, 16)] = jnp.zeros((16,), jnp.float32)
        return c

    lax.fori_loop(0, _CH, _zrow, 0)

    base = sid * _RPS
    for r in range(_RPS // _CH):
        pltpu.sync_copy(rows0, acc.at[pl.ds(base + r * _CH, _CH)])
    plsc.subcore_barrier()

    rows = (rows0, rows1)
    gsem = (gs0, gs1)
    ssem = (ss0, ss1)
    for h in range(_NCH // _HC):
        hb = wid * _NCH + h * _HC
        pltpu.sync_copy(src_hbm.at[pl.ds(hb, _HC)], idx_s)
        pltpu.sync_copy(dst_hbm.at[pl.ds(hb, _HC)], idx_d)
        gh = [None, None]
        sh = [None, None]
        gh[0] = pltpu.async_copy(ts_hbm.at[idx_s.at[0]], rows[0], gsem[0])
        for g in range(_HC):
            b = g & 1
            nb = b ^ 1
            gh[b].wait()
            if g + 1 < _HC:
                if sh[nb] is not None:
                    sh[nb].wait()
                gh[nb] = pltpu.async_copy(ts_hbm.at[idx_s.at[g + 1]],
                                          rows[nb], gsem[nb])
            sh[b] = pltpu.async_copy(rows[b], acc.at[idx_d.at[g]], ssem[b],
                                     add=True)
        sh[(_HC - 1) & 1].wait()
        if _HC >= 2:
            sh[(_HC - 2) & 1].wait()

    plsc.subcore_barrier()
    pltpu.sync_copy(acc.at[pl.ds(base, _RPS)],
                    out_hbm.at[pl.ds(cid * _NP + base, _RPS)])


@functools.cache
def _sc_calls():
    mesh = plsc.VectorSubcoreMesh(core_axis_name="c", subcore_axis_name="s")
    deg_call = pl.kernel(
        _deg_body,
        out_type=jax.ShapeDtypeStruct((_NC * _NP,), jnp.float32),
        mesh=mesh,
        scratch_types=[
            pltpu.VMEM((_NCH, _CH), jnp.int32),
            pltpu.VMEM((_CH,), jnp.float32),
            pltpu.VMEM((_RPS,), jnp.float32),
            pltpu.VMEM_SHARED((_NP,), jnp.float32),
            pltpu.SemaphoreType.DMA,
        ],
    )
    conv_call = pl.kernel(
        _conv_body,
        out_type=jax.ShapeDtypeStruct((_NC * _NP, _D), jnp.float32),
        mesh=mesh,
        scratch_types=[
            pltpu.VMEM((_HC, _CH), jnp.int32),
            pltpu.VMEM((_HC, _CH), jnp.int32),
            pltpu.VMEM((_CH, _D), jnp.float32),
            pltpu.VMEM((_CH, _D), jnp.float32),
            pltpu.VMEM_SHARED((_NP, _D), jnp.float32),
            pltpu.SemaphoreType.DMA,
            pltpu.SemaphoreType.DMA,
            pltpu.SemaphoreType.DMA,
            pltpu.SemaphoreType.DMA,
        ],
    )
    return deg_call, conv_call


def _k1_body(x_ref, win_ref, bin_ref, wc1_ref, d0_ref, d1_ref,
             ts_ref, dis_ref):
    h0 = jnp.maximum(
        jnp.dot(x_ref[...], win_ref[...], preferred_element_type=jnp.float32)
        + bin_ref[...], 0.0)
    deg = d0_ref[0] + d1_ref[0] + 1.0
    dis = lax.rsqrt(deg)
    dis_ref[...] = dis
    ts_ref[...] = jnp.dot(h0, wc1_ref[...],
                          preferred_element_type=jnp.float32) * dis


def _mid_body(a0_ref, a1_ref, ts_ref, dis_ref, bc_ref, g_ref, be_ref, w_ref,
              o_ref):
    dis = dis_ref[...]
    u = (a0_ref[0] + a1_ref[0] + ts_ref[...]) * dis + bc_ref[...]
    m = jnp.mean(u, axis=-1, keepdims=True)
    c = u - m
    v = jnp.mean(c * c, axis=-1, keepdims=True)
    h = jnp.maximum(c * lax.rsqrt(v + 1e-5) * g_ref[...] + be_ref[...], 0.0)
    o_ref[...] = jnp.dot(h, w_ref[...],
                         preferred_element_type=jnp.float32) * dis


def _fin_body(a0_ref, a1_ref, ts_ref, dis_ref, bc_ref, g_ref, be_ref, w_ref,
              bo_ref, o_ref):
    u = (a0_ref[0] + a1_ref[0] + ts_ref[...]) * dis_ref[...] + bc_ref[...]
    m = jnp.mean(u, axis=-1, keepdims=True)
    c = u - m
    v = jnp.mean(c * c, axis=-1, keepdims=True)
    h = jnp.maximum(c * lax.rsqrt(v + 1e-5) * g_ref[...] + be_ref[...], 0.0)
    o_ref[...] = jnp.dot(h, w_ref[...],
                         preferred_element_type=jnp.float32) + bo_ref[...]


_GRID = (_N // _RB,)

def _row_spec():
    return pl.BlockSpec((_RB, _D), lambda i: (i, 0))

def _w_spec():
    return pl.BlockSpec((_D, _D), lambda i: (0, 0))

def _b_spec():
    return pl.BlockSpec((1, _D), lambda i: (0, 0))

def _part_spec(c):
    return pl.BlockSpec((1, _RB, _D), lambda i, c=c: (c, i, 0))

def _deg_spec(c):
    return pl.BlockSpec((1, _RB, 1), lambda i, c=c: (c, i, 0))

def _dis_spec():
    return pl.BlockSpec((_RB, 1), lambda i: (i, 0))


_k1_call = pl.pallas_call(
    _k1_body,
    grid=_GRID,
    in_specs=[_row_spec(), _w_spec(), _b_spec(), _w_spec(),
              _deg_spec(0), _deg_spec(1)],
    out_specs=[_row_spec(), _dis_spec()],
    out_shape=[jax.ShapeDtypeStruct((_N, _D), jnp.float32),
               jax.ShapeDtypeStruct((_N, 1), jnp.float32)],
)

_mid_call = pl.pallas_call(
    _mid_body,
    grid=_GRID,
    in_specs=[_part_spec(0), _part_spec(1), _row_spec(), _dis_spec(),
              _b_spec(), _b_spec(), _b_spec(), _w_spec()],
    out_specs=_row_spec(),
    out_shape=jax.ShapeDtypeStruct((_N, _D), jnp.float32),
)

_fin_call = pl.pallas_call(
    _fin_body,
    grid=_GRID,
    in_specs=[_part_spec(0), _part_spec(1), _row_spec(), _dis_spec(),
              _b_spec(), _b_spec(), _b_spec(), _w_spec(), _b_spec()],
    out_specs=_row_spec(),
    out_shape=jax.ShapeDtypeStruct((_N, _D), jnp.float32),
)


def kernel(x, edge_index, W_in, b_in, W_c1, b_c1, g1, be1,
           W_c2, b_c2, g2, be2, W_out, b_out):
    src = edge_index[0].astype(jnp.int32)
    dst = edge_index[1].astype(jnp.int32)
    pad = _EPAD - _E
    srcp = jnp.concatenate([src, jnp.zeros((pad,), jnp.int32)]
                           ).reshape(_EPAD // _CH, _CH)
    dstp = jnp.concatenate([dst, jnp.full((pad,), _N, jnp.int32)]
                           ).reshape(_EPAD // _CH, _CH)

    deg_call, conv_call = _sc_calls()
    degp = deg_call(dstp).reshape(_NC, _NP, 1)
    ts1, dis = _k1_call(x, W_in, b_in.reshape(1, _D), W_c1, degp, degp)
    acc1 = conv_call(ts1, srcp, dstp).reshape(_NC, _NP, _D)
    ts2 = _mid_call(acc1, acc1, ts1, dis, b_c1.reshape(1, _D),
                    g1.reshape(1, _D), be1.reshape(1, _D), W_c2)
    acc2 = conv_call(ts2, srcp, dstp).reshape(_NC, _NP, _D)
    return _fin_call(acc2, acc2, ts2, dis, b_c2.reshape(1, _D),
                     g2.reshape(1, _D), be2.reshape(1, _D), W_out,
                     b_out.reshape(1, _D))


# speedup vs baseline: 12.1575x; 1.1603x over previous
import functools

import jax
import jax.numpy as jnp
from jax import lax
from jax.experimental import pallas as pl
from jax.experimental.pallas import tpu as pltpu
from jax.experimental.pallas import tpu_sc as plsc

_N = 10000
_D = 128
_E = 320000

_NC = 2
_NS = 16
_NW = _NC * _NS
_CH = 128
_NCH = (-(-_E // (_NW * _CH)) + 7) // 8 * 8
_EPAD = _NW * _CH * _NCH
_HC = _NCH // 2
_C0 = 120
_C1 = 40
_PST = _C0 + _C1
_NP = 10240
_RPS = _NP // _NS
_RB = 2000


def _deg_body(dst_hbm, out_hbm, idx_d, ones_v, zb, deg, dsem):
    cid = lax.axis_index("c")
    sid = lax.axis_index("s")
    wid = sid * _NC + cid

    def _z16(i, c):
        zb[pl.ds(i * 16, 16)] = jnp.zeros((16,), jnp.float32)
        return c

    lax.fori_loop(0, _RPS // 16, _z16, 0)
    for j in range(_CH // 16):
        ones_v[pl.ds(j * 16, 16)] = jnp.ones((16,), jnp.float32)

    base = sid * _RPS
    pltpu.sync_copy(zb, deg.at[pl.ds(base, _RPS)])
    pltpu.sync_copy(dst_hbm.at[pl.ds(wid * _NCH, _NCH)], idx_d)
    plsc.subcore_barrier()

    hs = []
    for g in range(_NCH):
        hs.append(pltpu.async_copy(ones_v, deg.at[idx_d.at[g]], dsem, add=True))
    for h in hs:
        h.wait()

    plsc.subcore_barrier()
    pltpu.sync_copy(deg.at[pl.ds(base, _RPS)],
                    out_hbm.at[pl.ds(cid * _NP + base, _RPS)])


def _conv_body(ts_hbm, src_hbm, dst_hbm, out_hbm,
               idx_s, idx_d, rows0, rows1, acc, gs0, gs1, ss0, ss1):
    cid = lax.axis_index("c")
    sid = lax.axis_index("s")
    wid = sid * _NC + cid

    def _zrow(i, c):
        for j in range(_D // 16):
            rows0[i, pl.ds(j * 16, 16)] = jnp.zeros((16,), jnp.float32)
        return c

    lax.fori_loop(0, _CH, _zrow, 0)

    base = sid * _RPS
    for r in range(_RPS // _CH):
        pltpu.sync_copy(rows0, acc.at[pl.ds(base + r * _CH, _CH)])
    plsc.subcore_barrier()

    rows = (rows0, rows1)
    gsem = (gs0, gs1)
    ssem = (ss0, ss1)

    def _edge_pipe(row0, nref):
        for h in range(nref):
            hb = row0 + h * _HC
            pltpu.sync_copy(src_hbm.at[pl.ds(hb, _HC)], idx_s)
            pltpu.sync_copy(dst_hbm.at[pl.ds(hb, _HC)], idx_d)
            gh = [None, None]
            sh = [None, None]
            gh[0] = pltpu.async_copy(ts_hbm.at[idx_s.at[0]], rows[0], gsem[0])
            for g in range(_HC):
                b = g & 1
                nb = b ^ 1
                gh[b].wait()
                if g + 1 < _HC:
                    if sh[nb] is not None:
                        sh[nb].wait()
                    gh[nb] = pltpu.async_copy(ts_hbm.at[idx_s.at[g + 1]],
                                              rows[nb], gsem[nb])
                sh[b] = pltpu.async_copy(rows[b], acc.at[idx_d.at[g]],
                                         ssem[b], add=True)
            sh[(_HC - 1) & 1].wait()
            if _HC >= 2:
                sh[(_HC - 2) & 1].wait()

    pair = sid * _PST

    @pl.when(cid == 0)
    def _():
        _edge_pipe(pair, _C0 // _HC)

    @pl.when(cid != 0)
    def _():
        _edge_pipe(pair + _C0, _C1 // _HC)

    plsc.subcore_barrier()
    pltpu.sync_copy(acc.at[pl.ds(base, _RPS)],
                    out_hbm.at[pl.ds(cid * _NP + base, _RPS)])


@functools.cache
def _sc_calls():
    mesh = plsc.VectorSubcoreMesh(core_axis_name="c", subcore_axis_name="s")
    deg_call = pl.kernel(
        _deg_body,
        out_type=jax.ShapeDtypeStruct((_NC * _NP,), jnp.float32),
        mesh=mesh,
        scratch_types=[
            pltpu.VMEM((_NCH, _CH), jnp.int32),
            pltpu.VMEM((_CH,), jnp.float32),
            pltpu.VMEM((_RPS,), jnp.float32),
            pltpu.VMEM_SHARED((_NP,), jnp.float32),
            pltpu.SemaphoreType.DMA,
        ],
    )
    conv_call = pl.kernel(
        _conv_body,
        out_type=jax.ShapeDtypeStruct((_NC * _NP, _D), jnp.float32),
        mesh=mesh,
        scratch_types=[
            pltpu.VMEM((_HC, _CH), jnp.int32),
            pltpu.VMEM((_HC, _CH), jnp.int32),
            pltpu.VMEM((_CH, _D), jnp.float32),
            pltpu.VMEM((_CH, _D), jnp.float32),
            pltpu.VMEM_SHARED((_NP, _D), jnp.float32),
            pltpu.SemaphoreType.DMA,
            pltpu.SemaphoreType.DMA,
            pltpu.SemaphoreType.DMA,
            pltpu.SemaphoreType.DMA,
        ],
    )
    return deg_call, conv_call


def _k1_body(x_ref, win_ref, bin_ref, wc1_ref, d0_ref, d1_ref,
             ts_ref, dis_ref):
    h0 = jnp.maximum(
        jnp.dot(x_ref[...], win_ref[...], preferred_element_type=jnp.float32)
        + bin_ref[...], 0.0)
    deg = d0_ref[0] + d1_ref[0] + 1.0
    dis = lax.rsqrt(deg)
    dis_ref[...] = dis
    ts_ref[...] = jnp.dot(h0, wc1_ref[...],
                          preferred_element_type=jnp.float32) * dis


def _mid_body(a0_ref, a1_ref, ts_ref, dis_ref, bc_ref, g_ref, be_ref, w_ref,
              o_ref):
    dis = dis_ref[...]
    u = (a0_ref[0] + a1_ref[0] + ts_ref[...]) * dis + bc_ref[...]
    m = jnp.mean(u, axis=-1, keepdims=True)
    c = u - m
    v = jnp.mean(c * c, axis=-1, keepdims=True)
    h = jnp.maximum(c * lax.rsqrt(v + 1e-5) * g_ref[...] + be_ref[...], 0.0)
    o_ref[...] = jnp.dot(h, w_ref[...],
                         preferred_element_type=jnp.float32) * dis


def _fin_body(a0_ref, a1_ref, ts_ref, dis_ref, bc_ref, g_ref, be_ref, w_ref,
              bo_ref, o_ref):
    u = (a0_ref[0] + a1_ref[0] + ts_ref[...]) * dis_ref[...] + bc_ref[...]
    m = jnp.mean(u, axis=-1, keepdims=True)
    c = u - m
    v = jnp.mean(c * c, axis=-1, keepdims=True)
    h = jnp.maximum(c * lax.rsqrt(v + 1e-5) * g_ref[...] + be_ref[...], 0.0)
    o_ref[...] = jnp.dot(h, w_ref[...],
                         preferred_element_type=jnp.float32) + bo_ref[...]


_GRID = (_N // _RB,)

def _row_spec():
    return pl.BlockSpec((_RB, _D), lambda i: (i, 0))

def _w_spec():
    return pl.BlockSpec((_D, _D), lambda i: (0, 0))

def _b_spec():
    return pl.BlockSpec((1, _D), lambda i: (0, 0))

def _part_spec(c):
    return pl.BlockSpec((1, _RB, _D), lambda i, c=c: (c, i, 0))

def _deg_spec(c):
    return pl.BlockSpec((1, _RB, 1), lambda i, c=c: (c, i, 0))

def _dis_spec():
    return pl.BlockSpec((_RB, 1), lambda i: (i, 0))


_k1_call = pl.pallas_call(
    _k1_body,
    grid=_GRID,
    in_specs=[_row_spec(), _w_spec(), _b_spec(), _w_spec(),
              _deg_spec(0), _deg_spec(1)],
    out_specs=[_row_spec(), _dis_spec()],
    out_shape=[jax.ShapeDtypeStruct((_N, _D), jnp.float32),
               jax.ShapeDtypeStruct((_N, 1), jnp.float32)],
)

_mid_call = pl.pallas_call(
    _mid_body,
    grid=_GRID,
    in_specs=[_part_spec(0), _part_spec(1), _row_spec(), _dis_spec(),
              _b_spec(), _b_spec(), _b_spec(), _w_spec()],
    out_specs=_row_spec(),
    out_shape=jax.ShapeDtypeStruct((_N, _D), jnp.float32),
)

_fin_call = pl.pallas_call(
    _fin_body,
    grid=_GRID,
    in_specs=[_part_spec(0), _part_spec(1), _row_spec(), _dis_spec(),
              _b_spec(), _b_spec(), _b_spec(), _w_spec(), _b_spec()],
    out_specs=_row_spec(),
    out_shape=jax.ShapeDtypeStruct((_N, _D), jnp.float32),
)


def kernel(x, edge_index, W_in, b_in, W_c1, b_c1, g1, be1,
           W_c2, b_c2, g2, be2, W_out, b_out):
    src = edge_index[0].astype(jnp.int32)
    dst = edge_index[1].astype(jnp.int32)
    pad = _EPAD - _E
    srcp = jnp.concatenate([src, jnp.zeros((pad,), jnp.int32)]
                           ).reshape(_EPAD // _CH, _CH)
    dstp = jnp.concatenate([dst, jnp.full((pad,), _N, jnp.int32)]
                           ).reshape(_EPAD // _CH, _CH)

    deg_call, conv_call = _sc_calls()
    degp = deg_call(dstp).reshape(_NC, _NP, 1)
    ts1, dis = _k1_call(x, W_in, b_in.reshape(1, _D), W_c1, degp, degp)
    acc1 = conv_call(ts1, srcp, dstp).reshape(_NC, _NP, _D)
    ts2 = _mid_call(acc1, acc1, ts1, dis, b_c1.reshape(1, _D),
                    g1.reshape(1, _D), be1.reshape(1, _D), W_c2)
    acc2 = conv_call(ts2, srcp, dstp).reshape(_NC, _NP, _D)
    return _fin_call(acc2, acc2, ts2, dis, b_c2.reshape(1, _D),
                     g2.reshape(1, _D), be2.reshape(1, _D), W_out,
                     b_out.reshape(1, _D))
